# baseline (speedup 1.0000x reference)
import functools

import jax
import jax.numpy as jnp
from jax import lax
from jax.experimental import pallas as pl
from jax.experimental.pallas import tpu as pltpu
from jax.experimental.pallas import tpu_sc as plsc

_N = 10000
_NP = 10240
_DIN = 128
_DH = 256
_DO = 128
_NEG = -1e30

_NC = 2
_NS = 16
_NW = _NC * _NS
_B = 64
_EBT = 164
_EBH = 82
_ET = _EBT * _B
_EP = _ET * _NW
_RPS = _NP // _NS

_f32 = jnp.float32
_i32 = jnp.int32


def _gat_pass(c, s, wid, sd_h, sd_v, slot0, slot1, cm_v, rows_v,
              acc_sh, s_sh, table_h, atts_h, attd_h, cmcols, acc_o, s_o):
    cs, cd = cmcols
    t = cm_v[cs] + cm_v[cd]
    cmax = jnp.maximum(t, 0.2 * t)

    def zsrc(r, carry):
        for j in range(8):
            rows_v[r, pl.ds(j * 16, 16)] = jnp.zeros((16,), _f32)
        slot0[2][r] = jnp.zeros((16,), _f32)
        return carry
    lax.fori_loop(0, _B, zsrc, 0)

    base = s * _RPS

    def zacc(k, carry):
        pltpu.sync_copy(rows_v, acc_sh.at[pl.ds(base + k * _B, _B)])
        if s_o is not None:
            pltpu.sync_copy(slot0[2], s_sh.at[pl.ds(base + k * _B, _B)])
        return carry
    lax.fori_loop(0, _RPS // _B, zacc, 0)
    plsc.subcore_barrier()

    def issue(b, slot):
        srcb, dstb, attsg, attdg, hpk, sem = slot
        for j in range(_B // 16):
            sl = pl.ds(j * 16, 16)
            v = sd_v[b, sl]
            srcb[0, sl] = lax.bitwise_and(v, 16383)
            dstb[0, sl] = lax.shift_right_logical(v, 14)
        pltpu.async_copy(atts_h.at[srcb.at[0]], attsg, sem)
        pltpu.async_copy(attd_h.at[dstb.at[0]], attdg, sem)
        pltpu.async_copy(table_h.at[srcb.at[0]], hpk, sem)

    def process(slot):
        srcb, dstb, attsg, attdg, hpk, sem = slot
        pltpu.make_async_copy(atts_h.at[srcb.at[0]], attsg, sem).wait()
        pltpu.make_async_copy(attd_h.at[dstb.at[0]], attdg, sem).wait()
        pltpu.make_async_copy(table_h.at[srcb.at[0]], hpk, sem).wait()

        def row_body(r, carry2):
            t16 = attsg[r] + attdg[r]
            al = jnp.maximum(t16, 0.2 * t16)
            ev = jnp.exp(al - cmax)
            for j in range(4):
                w = hpk[r, pl.ds(j * 16, 16)]
                lo = lax.bitcast_convert_type(lax.shift_left(w, 16), _f32)
                hi = lax.bitcast_convert_type(
                    lax.bitwise_and(w, -65536), _f32)
                rows_v[r, pl.ds(j * 16, 16)] = lo * ev
                rows_v[r, pl.ds(64 + j * 16, 16)] = hi * ev
            attsg[r] = ev
            return carry2
        lax.fori_loop(0, _B, row_body, 0)

        pltpu.sync_copy(rows_v, acc_sh.at[dstb.at[0]], add=True)
        if s_o is not None:
            pltpu.sync_copy(attsg, s_sh.at[dstb.at[0]], add=True)

    def half_body(hh, carry0):
        pltpu.sync_copy(sd_h.at[pl.ds(wid * _EBT + hh * _EBH, _EBH)], sd_v)
        issue(0, slot0)

        def pair_body(i, carry):
            issue(2 * i + 1, slot1)
            process(slot0)

            @pl.when(i + 1 < _EBH // 2)
            def _():
                issue(2 * i + 2, slot0)
            process(slot1)
            return carry
        lax.fori_loop(0, _EBH // 2, pair_body, 0)
        return carry0
    lax.fori_loop(0, 2, half_body, 0)
    plsc.subcore_barrier()

    def dump_body(k, carry):
        pltpu.sync_copy(acc_sh.at[pl.ds(base + k * _B, _B)],
                        acc_o.at[c, pl.ds(base + k * _B, _B)])
        if s_o is not None:
            pltpu.sync_copy(s_sh.at[pl.ds(base + k * _B, _B)],
                            s_o.at[c, pl.ds(base + k * _B, _B)])
        return carry
    lax.fori_loop(0, _RPS // _B, dump_body, 0)
    plsc.subcore_barrier()


_sc_mesh = plsc.VectorSubcoreMesh(core_axis_name="c", subcore_axis_name="s")

_acc_ty = jax.ShapeDtypeStruct((_NC, _NP, 128), _f32)
_s_ty = jax.ShapeDtypeStruct((_NC, _NP, 16), _f32)

_slot_scratch = [
    pltpu.VMEM((1, _B), _i32),
    pltpu.VMEM((1, _B), _i32),
    pltpu.VMEM((_B, 16), _f32),
    pltpu.VMEM((_B, 16), _f32),
    pltpu.VMEM((_B, 64), _i32),
    pltpu.SemaphoreType.DMA,
]

_sc_scratch = (
    [pltpu.VMEM((_EBH, _B), _i32)]
    + _slot_scratch + _slot_scratch
    + [
        pltpu.VMEM((8, 16), _f32),
        pltpu.VMEM((_B, 128), _f32),
        pltpu.VMEM_SHARED((_NP, 128), _f32),
        pltpu.VMEM_SHARED((_NP, 16), _f32),
    ]
)


_sc_params = pltpu.CompilerParams(use_tc_tiling_on_sc=False)


@functools.partial(
    pl.kernel, mesh=_sc_mesh,
    out_type=[_acc_ty, _acc_ty, _acc_ty, _acc_ty, _s_ty, _s_ty],
    scratch_types=_sc_scratch,
    compiler_params=_sc_params,
)
def _sc_layer0(sd_h, cm_h, asb_h, adb_h, asl_h, adl_h,
               hb0_h, hb1_h, hl0_h, hl1_h,
               accA0_o, accA1_o, accB0_o, accB1_o, sA_o, sB_o,
               sd_v, sb0, db0, ag0, dg0, hp0, sm0, sb1, db1, ag1, dg1, hp1,
               sm1, cm_v, rows_v, acc_sh, s_sh):
    c = lax.axis_index("c")
    s = lax.axis_index("s")
    wid = s * _NC + c
    pltpu.sync_copy(cm_h, cm_v)
    slot0 = (sb0, db0, ag0, dg0, hp0, sm0)
    slot1 = (sb1, db1, ag1, dg1, hp1, sm1)
    common = (c, s, wid, sd_h, sd_v, slot0, slot1, cm_v, rows_v,
              acc_sh, s_sh)
    _gat_pass(*common, hb0_h, asb_h, adb_h, (0, 1), accA0_o, sA_o)
    _gat_pass(*common, hb1_h, asb_h, adb_h, (0, 1), accA1_o, None)
    _gat_pass(*common, hl0_h, asl_h, adl_h, (2, 3), accB0_o, sB_o)
    _gat_pass(*common, hl1_h, asl_h, adl_h, (2, 3), accB1_o, None)


@functools.partial(
    pl.kernel, mesh=_sc_mesh,
    out_type=[_acc_ty, _acc_ty, _s_ty, _s_ty],
    scratch_types=_sc_scratch,
    compiler_params=_sc_params,
)
def _sc_layer1(sd_h, cm_h, asb_h, adb_h, asl_h, adl_h, hc_h, hd_h,
               accC_o, accD_o, sC_o, sD_o,
               sd_v, sb0, db0, ag0, dg0, hp0, sm0, sb1, db1, ag1, dg1, hp1,
               sm1, cm_v, rows_v, acc_sh, s_sh):
    c = lax.axis_index("c")
    s = lax.axis_index("s")
    wid = s * _NC + c
    pltpu.sync_copy(cm_h, cm_v)
    slot0 = (sb0, db0, ag0, dg0, hp0, sm0)
    slot1 = (sb1, db1, ag1, dg1, hp1, sm1)
    common = (c, s, wid, sd_h, sd_v, slot0, slot1, cm_v, rows_v,
              acc_sh, s_sh)
    _gat_pass(*common, hc_h, asb_h, adb_h, (0, 1), accC_o, sC_o)
    _gat_pass(*common, hd_h, asl_h, adl_h, (2, 3), accD_o, sD_o)


_BLK = 1024
_NBLK = _NP // _BLK


def _dot(a, b):
    return lax.dot(a, b, preferred_element_type=_f32)


def _pack2(lo, hi):
    bl = lax.bitcast_convert_type(lo, jnp.uint32)
    bh = lax.bitcast_convert_type(hi, jnp.uint32)
    rl = (bl + jnp.uint32(0x7FFF) + ((bl >> 16) & jnp.uint32(1))) >> 16
    rh = (bh + jnp.uint32(0x7FFF) + ((bh >> 16) & jnp.uint32(1))) >> 16
    return lax.bitcast_convert_type(rl | (rh << 16), _i32)


def _valid_mask16():
    rid = pl.program_id(0) * _BLK + lax.broadcasted_iota(_i32, (_BLK, 16), 0)
    return rid < _N


def _att16(scal, col, mask):
    rep = jnp.broadcast_to(scal[:, col:col + 1], (_BLK, 16))
    return jnp.where(mask, rep, _NEG)


def _accum_max(cm_ref, scal):
    m = jnp.max(scal, axis=0, keepdims=True)

    @pl.when(pl.program_id(0) == 0)
    def _():
        cm_ref[...] = m

    @pl.when(pl.program_id(0) != 0)
    def _():
        cm_ref[...] = jnp.maximum(cm_ref[...], m)


def _proj0_body(x_ref, w0_ref, a0_ref, b0w_ref, avb_ref, avl_ref,
                hb0_ref, hb1_ref, hl0_ref, hl1_ref,
                asb_ref, adb_ref, asl_ref, adl_ref, cm_ref):
    xb = x_ref[...]
    hb = _dot(xb, w0_ref[...])
    hl = _dot(_dot(xb, a0_ref[...]), b0w_ref[...])
    hb0_ref[...] = _pack2(hb[:, 0:64], hb[:, 64:128])
    hb1_ref[...] = _pack2(hb[:, 128:192], hb[:, 192:256])
    hl0_ref[...] = _pack2(hl[:, 0:64], hl[:, 64:128])
    hl1_ref[...] = _pack2(hl[:, 128:192], hl[:, 192:256])
    scal = _dot(hb, avb_ref[...]) + _dot(hl, avl_ref[...])
    mask = _valid_mask16()
    asb_ref[...] = _att16(scal, 0, mask)
    adb_ref[...] = _att16(scal, 1, mask)
    asl_ref[...] = _att16(scal, 2, mask)
    adl_ref[...] = _att16(scal, 3, mask)
    _accum_max(cm_ref, scal)


def _proj0(xp, w0t, a0t, b0t, avb, avl):
    full = lambda shape: pl.BlockSpec(shape, lambda i: (0, 0))
    rows = lambda width: pl.BlockSpec((_BLK, width), lambda i: (i, 0))
    return pl.pallas_call(
        _proj0_body,
        grid=(_NBLK,),
        in_specs=[rows(_DIN), full((_DIN, _DH)), full((_DIN, 32)),
                  full((32, _DH)), full((_DH, 8)), full((_DH, 8))],
        out_specs=[rows(64), rows(64), rows(64), rows(64),
                   rows(16), rows(16), rows(16), rows(16), full((1, 8))],
        out_shape=[jax.ShapeDtypeStruct((_NP, 64), _i32)] * 4
        + [jax.ShapeDtypeStruct((_NP, 16), _f32)] * 4
        + [jax.ShapeDtypeStruct((1, 8), _f32)],
    )(xp, w0t, a0t, b0t, avb, avl)


def _mid_body(a00_ref, a01_ref, b00_ref, b01_ref, sa_ref, sb_ref,
              b0_ref, bl0_ref, w1_ref, a1_ref, b1w_ref, avb_ref, avl_ref,
              hc_ref, hd_ref, asb_ref, adb_ref, asl_ref, adl_ref, cm_ref):
    ra = sa_ref[0] + sa_ref[1]
    rb = sb_ref[0] + sb_ref[1]
    recA = 1.0 / (jnp.broadcast_to(ra[:, 0:1], (_BLK, 128)) + 1e-16)
    recB = 1.0 / (jnp.broadcast_to(rb[:, 0:1], (_BLK, 128)) + 1e-16)
    bias0 = b0_ref[0:1, 0:128] + bl0_ref[0:1, 0:128]
    bias1 = b0_ref[0:1, 128:256] + bl0_ref[0:1, 128:256]
    x1c0 = (a00_ref[0] + a00_ref[1]) * recA + (b00_ref[0] + b00_ref[1]) * recB + bias0
    x1c1 = (a01_ref[0] + a01_ref[1]) * recA + (b01_ref[0] + b01_ref[1]) * recB + bias1
    w1 = w1_ref[...]
    a1 = a1_ref[...]
    hc = _dot(x1c0, w1[:128]) + _dot(x1c1, w1[128:])
    hd = _dot(_dot(x1c0, a1[:128]) + _dot(x1c1, a1[128:]), b1w_ref[...])
    hc_ref[...] = _pack2(hc[:, 0:64], hc[:, 64:128])
    hd_ref[...] = _pack2(hd[:, 0:64], hd[:, 64:128])
    scal = _dot(hc, avb_ref[...]) + _dot(hd, avl_ref[...])
    mask = _valid_mask16()
    asb_ref[...] = _att16(scal, 0, mask)
    adb_ref[...] = _att16(scal, 1, mask)
    asl_ref[...] = _att16(scal, 2, mask)
    adl_ref[...] = _att16(scal, 3, mask)
    _accum_max(cm_ref, scal)


def _mid(a00, a01, b00, b01, sa, sb, b0r, bl0r, w1t, a1t, b1t, avb, avl):
    acc = pl.BlockSpec((_NC, _BLK, 128), lambda i: (0, i, 0))
    sden = pl.BlockSpec((_NC, _BLK, 16), lambda i: (0, i, 0))
    full = lambda shape: pl.BlockSpec(shape, lambda i: (0, 0))
    rows = lambda width: pl.BlockSpec((_BLK, width), lambda i: (i, 0))
    return pl.pallas_call(
        _mid_body,
        grid=(_NBLK,),
        in_specs=[acc, acc, acc, acc, sden, sden,
                  full((1, _DH)), full((1, _DH)), full((_DH, 128)),
                  full((_DH, 32)), full((32, 128)),
                  full((128, 8)), full((128, 8))],
        out_specs=[rows(64), rows(64),
                   rows(16), rows(16), rows(16), rows(16), full((1, 8))],
        out_shape=[jax.ShapeDtypeStruct((_NP, 64), _i32)] * 2
        + [jax.ShapeDtypeStruct((_NP, 16), _f32)] * 4
        + [jax.ShapeDtypeStruct((1, 8), _f32)],
    )(a00, a01, b00, b01, sa, sb, b0r, bl0r, w1t, a1t, b1t, avb, avl)


def _fin_body(ac_ref, ad_ref, sc_ref, sd_ref, b1_ref, bl1_ref,
              out_ref, e1_ref, e2_ref):
    rc = sc_ref[0] + sc_ref[1]
    rd = sd_ref[0] + sd_ref[1]
    recC = 1.0 / (jnp.broadcast_to(rc[:, 0:1], (_BLK, 128)) + 1e-16)
    recD = 1.0 / (jnp.broadcast_to(rd[:, 0:1], (_BLK, 128)) + 1e-16)
    e1 = (ac_ref[0] + ac_ref[1]) * recC + b1_ref[0:1, :]
    e2 = (ad_ref[0] + ad_ref[1]) * recD + bl1_ref[0:1, :]
    e1_ref[...] = e1
    e2_ref[...] = e2
    out_ref[...] = e1 + e2


def _fin(accC, accD, sC, sD, b1r, bl1r):
    acc = pl.BlockSpec((_NC, _BLK, 128), lambda i: (0, i, 0))
    sden = pl.BlockSpec((_NC, _BLK, 16), lambda i: (0, i, 0))
    full = lambda shape: pl.BlockSpec(shape, lambda i: (0, 0))
    rows = pl.BlockSpec((_BLK, 128), lambda i: (i, 0))
    return pl.pallas_call(
        _fin_body,
        grid=(_NBLK,),
        in_specs=[acc, acc, sden, sden, full((1, 128)), full((1, 128))],
        out_specs=[rows, rows, rows],
        out_shape=[jax.ShapeDtypeStruct((_NP, 128), _f32)] * 3,
    )(accC, accD, sC, sD, b1r, bl1r)


def kernel(x, edge_index, W0, a_s0, a_d0, b0, W1, a_s1, a_d1, b1,
           A0, B0, a_sl0, a_dl0, bl0, A1, B1, a_sl1, a_dl1, bl1):
    n = _N
    loop = jnp.arange(n, dtype=edge_index.dtype)
    src = jnp.concatenate([edge_index[0], loop])
    dst = jnp.concatenate([edge_index[1], loop])
    pad = _EP - src.shape[0]
    src = jnp.concatenate([src, jnp.zeros((pad,), _i32)])
    dst = jnp.concatenate([dst, jnp.full((pad,), n, _i32)])
    sd3 = ((dst << 14) | src).reshape(_NW * _EBT, _B)

    xp = jnp.pad(x, ((0, _NP - n), (0, 0)))
    av0b = jnp.zeros((_DH, 8), _f32).at[:, 0].set(a_s0).at[:, 1].set(a_d0)
    av0l = jnp.zeros((_DH, 8), _f32).at[:, 2].set(a_sl0).at[:, 3].set(a_dl0)
    av1b = jnp.zeros((_DO, 8), _f32).at[:, 0].set(a_s1).at[:, 1].set(a_d1)
    av1l = jnp.zeros((_DO, 8), _f32).at[:, 2].set(a_sl1).at[:, 3].set(a_dl1)

    (hb0, hb1, hl0, hl1, asb0, adb0, asl0, adl0, cm0) = _proj0(
        xp, W0.T, A0.T, B0.T, av0b, av0l)
    cm0p = jnp.broadcast_to(cm0[0][:, None], (8, 16))

    accA0, accA1, accB0, accB1, sA, sB = _sc_layer0(
        sd3, cm0p, asb0, adb0, asl0, adl0, hb0, hb1, hl0, hl1)

    (hc, hd, asb1, adb1, asl1, adl1, cm1) = _mid(
        accA0, accA1, accB0, accB1, sA, sB,
        b0[None, :], bl0[None, :], W1.T, A1.T, B1.T, av1b, av1l)
    cm1p = jnp.broadcast_to(cm1[0][:, None], (8, 16))

    accC, accD, sC, sD = _sc_layer1(sd3, cm1p, asb1, adb1, asl1, adl1, hc, hd)

    out, emb1, emb2 = _fin(accC, accD, sC, sD, b1[None, :], bl1[None, :])
    return (out[:n], emb1[:n], emb2[:n])


# speedup vs baseline: 1.1055x; 1.1055x over previous
import functools

import jax
import jax.numpy as jnp
from jax import lax
from jax.experimental import pallas as pl
from jax.experimental.pallas import tpu as pltpu
from jax.experimental.pallas import tpu_sc as plsc

_N = 10000
_NP = 10240
_DIN = 128
_DH = 256
_DO = 128
_NEG = -1e30

_NC = 2
_NS = 16
_NW = _NC * _NS
_B = 64
_EBT = 164
_EBH = 82
_ET = _EBT * _B
_EP = _ET * _NW
_RPS = _NP // _NS

_f32 = jnp.float32
_i32 = jnp.int32


def _gat_pass(c, s, wid, sd_h, sd_v, slot0, slot1, cm_v,
              acc_sh, s_sh, table_h, atts_h, attd_h, cmcols, acc_o, s_o):
    cs, cd = cmcols
    t = cm_v[cs] + cm_v[cd]
    cmax = jnp.maximum(t, 0.2 * t)

    def zsrc(r, carry):
        for j in range(8):
            slot0[6][r, pl.ds(j * 16, 16)] = jnp.zeros((16,), _f32)
        slot0[2][r] = jnp.zeros((16,), _f32)
        return carry
    lax.fori_loop(0, _B, zsrc, 0)

    base = s * _RPS

    def zacc(k, carry):
        pltpu.sync_copy(slot0[6], acc_sh.at[pl.ds(base + k * _B, _B)])
        if s_o is not None:
            pltpu.sync_copy(slot0[2], s_sh.at[pl.ds(base + k * _B, _B)])
        return carry
    lax.fori_loop(0, _RPS // _B, zacc, 0)
    plsc.subcore_barrier()

    def issue(b, slot):
        srcb, dstb, attsg, attdg, hpk, sem = slot[:6]
        for j in range(_B // 16):
            sl = pl.ds(j * 16, 16)
            v = sd_v[b, sl]
            srcb[0, sl] = lax.bitwise_and(v, 16383)
            dstb[0, sl] = lax.shift_right_logical(v, 14)
        pltpu.async_copy(atts_h.at[srcb.at[0]], attsg, sem)
        pltpu.async_copy(attd_h.at[dstb.at[0]], attdg, sem)
        pltpu.async_copy(table_h.at[srcb.at[0]], hpk, sem)

    def process(slot, first):
        srcb, dstb, attsg, attdg, hpk, sem, rows, dsts, scsem = slot
        pltpu.make_async_copy(atts_h.at[srcb.at[0]], attsg, sem).wait()
        pltpu.make_async_copy(attd_h.at[dstb.at[0]], attdg, sem).wait()
        pltpu.make_async_copy(table_h.at[srcb.at[0]], hpk, sem).wait()

        @pl.when(jnp.logical_not(first))
        def _():
            pltpu.make_async_copy(
                rows, acc_sh.at[dsts.at[0]], scsem).wait()

        def row_body(r, carry2):
            t16 = attsg[r] + attdg[r]
            al = jnp.maximum(t16, 0.2 * t16)
            ev = jnp.exp(al - cmax)
            for j in range(4):
                w = hpk[r, pl.ds(j * 16, 16)]
                lo = lax.bitcast_convert_type(lax.shift_left(w, 16), _f32)
                hi = lax.bitcast_convert_type(
                    lax.bitwise_and(w, -65536), _f32)
                rows[r, pl.ds(j * 16, 16)] = lo * ev
                rows[r, pl.ds(64 + j * 16, 16)] = hi * ev
            attsg[r] = ev
            return carry2
        lax.fori_loop(0, _B, row_body, 0)

        for j in range(_B // 16):
            sl = pl.ds(j * 16, 16)
            dsts[0, sl] = dstb[0, sl]

        pltpu.async_copy(rows, acc_sh.at[dsts.at[0]], scsem, add=True)
        if s_o is not None:
            pltpu.sync_copy(attsg, s_sh.at[dsts.at[0]], add=True)

    def half_body(hh, carry0):
        pltpu.sync_copy(sd_h.at[pl.ds(wid * _EBT + hh * _EBH, _EBH)], sd_v)
        issue(0, slot0)

        def pair_body(i, carry):
            first = jnp.logical_and(hh == 0, i == 0)
            issue(2 * i + 1, slot1)
            process(slot0, first)

            @pl.when(i + 1 < _EBH // 2)
            def _():
                issue(2 * i + 2, slot0)
            process(slot1, first)
            return carry
        lax.fori_loop(0, _EBH // 2, pair_body, 0)
        return carry0
    lax.fori_loop(0, 2, half_body, 0)

    for sl in (slot0, slot1):
        pltpu.make_async_copy(sl[6], acc_sh.at[sl[7].at[0]], sl[8]).wait()
    plsc.subcore_barrier()

    def dump_body(k, carry):
        pltpu.sync_copy(acc_sh.at[pl.ds(base + k * _B, _B)],
                        acc_o.at[c, pl.ds(base + k * _B, _B)])
        if s_o is not None:
            pltpu.sync_copy(s_sh.at[pl.ds(base + k * _B, _B)],
                            s_o.at[c, pl.ds(base + k * _B, _B)])
        return carry
    lax.fori_loop(0, _RPS // _B, dump_body, 0)
    plsc.subcore_barrier()


_sc_mesh = plsc.VectorSubcoreMesh(core_axis_name="c", subcore_axis_name="s")

_acc_ty = jax.ShapeDtypeStruct((_NC, _NP, 128), _f32)
_s_ty = jax.ShapeDtypeStruct((_NC, _NP, 16), _f32)

_slot_scratch = [
    pltpu.VMEM((1, _B), _i32),
    pltpu.VMEM((1, _B), _i32),
    pltpu.VMEM((_B, 16), _f32),
    pltpu.VMEM((_B, 16), _f32),
    pltpu.VMEM((_B, 64), _i32),
    pltpu.SemaphoreType.DMA,
    pltpu.VMEM((_B, 128), _f32),
    pltpu.VMEM((1, _B), _i32),
    pltpu.SemaphoreType.DMA,
]

_sc_scratch = (
    [pltpu.VMEM((_EBH, _B), _i32)]
    + _slot_scratch + _slot_scratch
    + [
        pltpu.VMEM((8, 16), _f32),
        pltpu.VMEM_SHARED((_NP, 128), _f32),
        pltpu.VMEM_SHARED((_NP, 16), _f32),
    ]
)


_sc_params = pltpu.CompilerParams(use_tc_tiling_on_sc=False)


@functools.partial(
    pl.kernel, mesh=_sc_mesh,
    out_type=[_acc_ty, _acc_ty, _acc_ty, _acc_ty, _s_ty, _s_ty],
    scratch_types=_sc_scratch,
    compiler_params=_sc_params,
)
def _sc_layer0(sd_h, cm_h, asb_h, adb_h, asl_h, adl_h,
               hb0_h, hb1_h, hl0_h, hl1_h,
               accA0_o, accA1_o, accB0_o, accB1_o, sA_o, sB_o,
               sd_v, sb0, db0, ag0, dg0, hp0, sm0, rw0, ds0, cs0,
               sb1, db1, ag1, dg1, hp1, sm1, rw1, ds1, cs1,
               cm_v, acc_sh, s_sh):
    c = lax.axis_index("c")
    s = lax.axis_index("s")
    wid = s * _NC + c
    pltpu.sync_copy(cm_h, cm_v)
    slot0 = (sb0, db0, ag0, dg0, hp0, sm0, rw0, ds0, cs0)
    slot1 = (sb1, db1, ag1, dg1, hp1, sm1, rw1, ds1, cs1)
    common = (c, s, wid, sd_h, sd_v, slot0, slot1, cm_v,
              acc_sh, s_sh)
    _gat_pass(*common, hb0_h, asb_h, adb_h, (0, 1), accA0_o, sA_o)
    _gat_pass(*common, hb1_h, asb_h, adb_h, (0, 1), accA1_o, None)
    _gat_pass(*common, hl0_h, asl_h, adl_h, (2, 3), accB0_o, sB_o)
    _gat_pass(*common, hl1_h, asl_h, adl_h, (2, 3), accB1_o, None)


@functools.partial(
    pl.kernel, mesh=_sc_mesh,
    out_type=[_acc_ty, _acc_ty, _s_ty, _s_ty],
    scratch_types=_sc_scratch,
    compiler_params=_sc_params,
)
def _sc_layer1(sd_h, cm_h, asb_h, adb_h, asl_h, adl_h, hc_h, hd_h,
               accC_o, accD_o, sC_o, sD_o,
               sd_v, sb0, db0, ag0, dg0, hp0, sm0, rw0, ds0, cs0,
               sb1, db1, ag1, dg1, hp1, sm1, rw1, ds1, cs1,
               cm_v, acc_sh, s_sh):
    c = lax.axis_index("c")
    s = lax.axis_index("s")
    wid = s * _NC + c
    pltpu.sync_copy(cm_h, cm_v)
    slot0 = (sb0, db0, ag0, dg0, hp0, sm0, rw0, ds0, cs0)
    slot1 = (sb1, db1, ag1, dg1, hp1, sm1, rw1, ds1, cs1)
    common = (c, s, wid, sd_h, sd_v, slot0, slot1, cm_v,
              acc_sh, s_sh)
    _gat_pass(*common, hc_h, asb_h, adb_h, (0, 1), accC_o, sC_o)
    _gat_pass(*common, hd_h, asl_h, adl_h, (2, 3), accD_o, sD_o)


_BLK = 1024
_NBLK = _NP // _BLK


def _dot(a, b):
    return lax.dot(a, b, preferred_element_type=_f32)


def _pack2(lo, hi):
    bl = lax.bitcast_convert_type(lo, jnp.uint32)
    bh = lax.bitcast_convert_type(hi, jnp.uint32)
    rl = (bl + jnp.uint32(0x7FFF) + ((bl >> 16) & jnp.uint32(1))) >> 16
    rh = (bh + jnp.uint32(0x7FFF) + ((bh >> 16) & jnp.uint32(1))) >> 16
    return lax.bitcast_convert_type(rl | (rh << 16), _i32)


def _valid_mask16():
    rid = pl.program_id(0) * _BLK + lax.broadcasted_iota(_i32, (_BLK, 16), 0)
    return rid < _N


def _att16(scal, col, mask):
    rep = jnp.broadcast_to(scal[:, col:col + 1], (_BLK, 16))
    return jnp.where(mask, rep, _NEG)


def _accum_max(cm_ref, scal):
    m = jnp.max(scal, axis=0, keepdims=True)

    @pl.when(pl.program_id(0) == 0)
    def _():
        cm_ref[...] = m

    @pl.when(pl.program_id(0) != 0)
    def _():
        cm_ref[...] = jnp.maximum(cm_ref[...], m)


def _proj0_body(x_ref, w0_ref, a0_ref, b0w_ref, avb_ref, avl_ref,
                hb0_ref, hb1_ref, hl0_ref, hl1_ref,
                asb_ref, adb_ref, asl_ref, adl_ref, cm_ref):
    xb = x_ref[...]
    hb = _dot(xb, w0_ref[...])
    hl = _dot(_dot(xb, a0_ref[...]), b0w_ref[...])
    hb0_ref[...] = _pack2(hb[:, 0:64], hb[:, 64:128])
    hb1_ref[...] = _pack2(hb[:, 128:192], hb[:, 192:256])
    hl0_ref[...] = _pack2(hl[:, 0:64], hl[:, 64:128])
    hl1_ref[...] = _pack2(hl[:, 128:192], hl[:, 192:256])
    scal = _dot(hb, avb_ref[...]) + _dot(hl, avl_ref[...])
    mask = _valid_mask16()
    asb_ref[...] = _att16(scal, 0, mask)
    adb_ref[...] = _att16(scal, 1, mask)
    asl_ref[...] = _att16(scal, 2, mask)
    adl_ref[...] = _att16(scal, 3, mask)
    _accum_max(cm_ref, scal)


def _proj0(xp, w0t, a0t, b0t, avb, avl):
    full = lambda shape: pl.BlockSpec(shape, lambda i: (0, 0))
    rows = lambda width: pl.BlockSpec((_BLK, width), lambda i: (i, 0))
    return pl.pallas_call(
        _proj0_body,
        grid=(_NBLK,),
        in_specs=[rows(_DIN), full((_DIN, _DH)), full((_DIN, 32)),
                  full((32, _DH)), full((_DH, 8)), full((_DH, 8))],
        out_specs=[rows(64), rows(64), rows(64), rows(64),
                   rows(16), rows(16), rows(16), rows(16), full((1, 8))],
        out_shape=[jax.ShapeDtypeStruct((_NP, 64), _i32)] * 4
        + [jax.ShapeDtypeStruct((_NP, 16), _f32)] * 4
        + [jax.ShapeDtypeStruct((1, 8), _f32)],
    )(xp, w0t, a0t, b0t, avb, avl)


def _mid_body(a00_ref, a01_ref, b00_ref, b01_ref, sa_ref, sb_ref,
              b0_ref, bl0_ref, w1_ref, a1_ref, b1w_ref, avb_ref, avl_ref,
              hc_ref, hd_ref, asb_ref, adb_ref, asl_ref, adl_ref, cm_ref):
    ra = sa_ref[0] + sa_ref[1]
    rb = sb_ref[0] + sb_ref[1]
    recA = 1.0 / (jnp.broadcast_to(ra[:, 0:1], (_BLK, 128)) + 1e-16)
    recB = 1.0 / (jnp.broadcast_to(rb[:, 0:1], (_BLK, 128)) + 1e-16)
    bias0 = b0_ref[0:1, 0:128] + bl0_ref[0:1, 0:128]
    bias1 = b0_ref[0:1, 128:256] + bl0_ref[0:1, 128:256]
    x1c0 = (a00_ref[0] + a00_ref[1]) * recA + (b00_ref[0] + b00_ref[1]) * recB + bias0
    x1c1 = (a01_ref[0] + a01_ref[1]) * recA + (b01_ref[0] + b01_ref[1]) * recB + bias1
    w1 = w1_ref[...]
    a1 = a1_ref[...]
    hc = _dot(x1c0, w1[:128]) + _dot(x1c1, w1[128:])
    hd = _dot(_dot(x1c0, a1[:128]) + _dot(x1c1, a1[128:]), b1w_ref[...])
    hc_ref[...] = _pack2(hc[:, 0:64], hc[:, 64:128])
    hd_ref[...] = _pack2(hd[:, 0:64], hd[:, 64:128])
    scal = _dot(hc, avb_ref[...]) + _dot(hd, avl_ref[...])
    mask = _valid_mask16()
    asb_ref[...] = _att16(scal, 0, mask)
    adb_ref[...] = _att16(scal, 1, mask)
    asl_ref[...] = _att16(scal, 2, mask)
    adl_ref[...] = _att16(scal, 3, mask)
    _accum_max(cm_ref, scal)


def _mid(a00, a01, b00, b01, sa, sb, b0r, bl0r, w1t, a1t, b1t, avb, avl):
    acc = pl.BlockSpec((_NC, _BLK, 128), lambda i: (0, i, 0))
    sden = pl.BlockSpec((_NC, _BLK, 16), lambda i: (0, i, 0))
    full = lambda shape: pl.BlockSpec(shape, lambda i: (0, 0))
    rows = lambda width: pl.BlockSpec((_BLK, width), lambda i: (i, 0))
    return pl.pallas_call(
        _mid_body,
        grid=(_NBLK,),
        in_specs=[acc, acc, acc, acc, sden, sden,
                  full((1, _DH)), full((1, _DH)), full((_DH, 128)),
                  full((_DH, 32)), full((32, 128)),
                  full((128, 8)), full((128, 8))],
        out_specs=[rows(64), rows(64),
                   rows(16), rows(16), rows(16), rows(16), full((1, 8))],
        out_shape=[jax.ShapeDtypeStruct((_NP, 64), _i32)] * 2
        + [jax.ShapeDtypeStruct((_NP, 16), _f32)] * 4
        + [jax.ShapeDtypeStruct((1, 8), _f32)],
    )(a00, a01, b00, b01, sa, sb, b0r, bl0r, w1t, a1t, b1t, avb, avl)


def _fin_body(ac_ref, ad_ref, sc_ref, sd_ref, b1_ref, bl1_ref,
              out_ref, e1_ref, e2_ref):
    rc = sc_ref[0] + sc_ref[1]
    rd = sd_ref[0] + sd_ref[1]
    recC = 1.0 / (jnp.broadcast_to(rc[:, 0:1], (_BLK, 128)) + 1e-16)
    recD = 1.0 / (jnp.broadcast_to(rd[:, 0:1], (_BLK, 128)) + 1e-16)
    e1 = (ac_ref[0] + ac_ref[1]) * recC + b1_ref[0:1, :]
    e2 = (ad_ref[0] + ad_ref[1]) * recD + bl1_ref[0:1, :]
    e1_ref[...] = e1
    e2_ref[...] = e2
    out_ref[...] = e1 + e2


def _fin(accC, accD, sC, sD, b1r, bl1r):
    acc = pl.BlockSpec((_NC, _BLK, 128), lambda i: (0, i, 0))
    sden = pl.BlockSpec((_NC, _BLK, 16), lambda i: (0, i, 0))
    full = lambda shape: pl.BlockSpec(shape, lambda i: (0, 0))
    rows = pl.BlockSpec((_BLK, 128), lambda i: (i, 0))
    return pl.pallas_call(
        _fin_body,
        grid=(_NBLK,),
        in_specs=[acc, acc, sden, sden, full((1, 128)), full((1, 128))],
        out_specs=[rows, rows, rows],
        out_shape=[jax.ShapeDtypeStruct((_NP, 128), _f32)] * 3,
    )(accC, accD, sC, sD, b1r, bl1r)


def kernel(x, edge_index, W0, a_s0, a_d0, b0, W1, a_s1, a_d1, b1,
           A0, B0, a_sl0, a_dl0, bl0, A1, B1, a_sl1, a_dl1, bl1):
    n = _N
    loop = jnp.arange(n, dtype=edge_index.dtype)
    src = jnp.concatenate([edge_index[0], loop])
    dst = jnp.concatenate([edge_index[1], loop])
    pad = _EP - src.shape[0]
    src = jnp.concatenate([src, jnp.zeros((pad,), _i32)])
    dst = jnp.concatenate([dst, jnp.full((pad,), n, _i32)])
    sd3 = ((dst << 14) | src).reshape(_NW * _EBT, _B)

    xp = jnp.pad(x, ((0, _NP - n), (0, 0)))
    av0b = jnp.zeros((_DH, 8), _f32).at[:, 0].set(a_s0).at[:, 1].set(a_d0)
    av0l = jnp.zeros((_DH, 8), _f32).at[:, 2].set(a_sl0).at[:, 3].set(a_dl0)
    av1b = jnp.zeros((_DO, 8), _f32).at[:, 0].set(a_s1).at[:, 1].set(a_d1)
    av1l = jnp.zeros((_DO, 8), _f32).at[:, 2].set(a_sl1).at[:, 3].set(a_dl1)

    (hb0, hb1, hl0, hl1, asb0, adb0, asl0, adl0, cm0) = _proj0(
        xp, W0.T, A0.T, B0.T, av0b, av0l)
    cm0p = jnp.broadcast_to(cm0[0][:, None], (8, 16))

    accA0, accA1, accB0, accB1, sA, sB = _sc_layer0(
        sd3, cm0p, asb0, adb0, asl0, adl0, hb0, hb1, hl0, hl1)

    (hc, hd, asb1, adb1, asl1, adl1, cm1) = _mid(
        accA0, accA1, accB0, accB1, sA, sB,
        b0[None, :], bl0[None, :], W1.T, A1.T, B1.T, av1b, av1l)
    cm1p = jnp.broadcast_to(cm1[0][:, None], (8, 16))

    accC, accD, sC, sD = _sc_layer1(sd3, cm1p, asb1, adb1, asl1, adl1, hc, hd)

    out, emb1, emb2 = _fin(accC, accD, sC, sD, b1[None, :], bl1[None, :])
    return (out[:n], emb1[:n], emb2[:n])
